# Initial kernel scaffold; baseline (speedup 1.0000x reference)
import functools

import jax
import jax.numpy as jnp
from jax import lax
from jax.experimental import pallas as pl
from jax.experimental.pallas import tpu as pltpu
from jax.experimental.pallas import tpu_sc as plsc

N = 10000
F = 128
F_OUT = 64
E = 320000
NC, NS = 2, 16
NW = NC * NS
EPW = E // NW
CHUNK = 80
NCHUNK = EPW // CHUNK
ROWS_PT = N // NS
DEG_W = 16


def _mesh():
    return plsc.VectorSubcoreMesh(core_axis_name="c", subcore_axis_name="s")


@functools.partial(
    pl.kernel,
    out_type=jax.ShapeDtypeStruct((NC, N, DEG_W), jnp.float32),
    mesh=_mesh(),
    scratch_types=[
        pltpu.VMEM_SHARED((N, DEG_W), jnp.float32),
        pltpu.VMEM((NCHUNK, CHUNK), jnp.int32),
        pltpu.VMEM((CHUNK, DEG_W), jnp.float32),
    ],
)
def _deg_kernel(dst_hbm, z16_hbm, ones_hbm, out_hbm, acc, dstb, ones):
    c = lax.axis_index("c")
    s = lax.axis_index("s")
    wid = s * NC + c
    pltpu.sync_copy(z16_hbm, acc.at[pl.ds(s * ROWS_PT, ROWS_PT)])
    pltpu.sync_copy(ones_hbm, ones)
    pltpu.sync_copy(dst_hbm.at[wid], dstb)
    plsc.subcore_barrier()

    def body(j, carry):
        pltpu.sync_copy(ones, acc.at[dstb.at[j]], add=True)
        return carry

    lax.fori_loop(0, NCHUNK, body, 0)
    plsc.subcore_barrier()
    pltpu.sync_copy(acc.at[pl.ds(s * ROWS_PT, ROWS_PT)],
                    out_hbm.at[c, pl.ds(s * ROWS_PT, ROWS_PT)])


@functools.partial(
    pl.kernel,
    out_type=jax.ShapeDtypeStruct((NC, N, F), jnp.float32),
    mesh=_mesh(),
    scratch_types=[
        pltpu.VMEM_SHARED((N, F), jnp.float32),
        pltpu.VMEM((NCHUNK, CHUNK), jnp.int32),
        pltpu.VMEM((NCHUNK, CHUNK), jnp.int32),
        pltpu.VMEM((CHUNK, F), jnp.float32),
        pltpu.VMEM((CHUNK, F), jnp.float32),
        pltpu.SemaphoreType.DMA,
        pltpu.SemaphoreType.DMA,
    ],
)
def _agg_kernel(y_hbm, src_hbm, dst_hbm, z128_hbm, out_hbm,
                acc, srcb, dstb, rows0, rows1, sem0, sem1):
    c = lax.axis_index("c")
    s = lax.axis_index("s")
    wid = s * NC + c
    pltpu.sync_copy(z128_hbm, acc.at[pl.ds(s * ROWS_PT, ROWS_PT)])
    pltpu.sync_copy(src_hbm.at[wid], srcb)
    pltpu.sync_copy(dst_hbm.at[wid], dstb)
    plsc.subcore_barrier()

    pltpu.async_copy(y_hbm.at[srcb.at[0]], rows0, sem0)

    def body(i, carry):
        jo = 2 * i
        pltpu.async_copy(y_hbm.at[srcb.at[jo + 1]], rows1, sem1)
        pltpu.make_async_copy(y_hbm.at[srcb.at[jo]], rows0, sem0).wait()
        pltpu.sync_copy(rows0, acc.at[dstb.at[jo]], add=True)
        pltpu.async_copy(y_hbm.at[srcb.at[jo + 2]], rows0, sem0)
        pltpu.make_async_copy(y_hbm.at[srcb.at[jo + 1]], rows1, sem1).wait()
        pltpu.sync_copy(rows1, acc.at[dstb.at[jo + 1]], add=True)
        return carry

    lax.fori_loop(0, (NCHUNK - 1) // 2, body, 0)
    pltpu.make_async_copy(y_hbm.at[srcb.at[NCHUNK - 1]], rows0, sem0).wait()
    pltpu.sync_copy(rows0, acc.at[dstb.at[NCHUNK - 1]], add=True)
    plsc.subcore_barrier()
    pltpu.sync_copy(acc.at[pl.ds(s * ROWS_PT, ROWS_PT)],
                    out_hbm.at[c, pl.ds(s * ROWS_PT, ROWS_PT)])


_BM = 1000


def _dis_from(deg_ref):
    deg = deg_ref[0, :, 0:1] + deg_ref[1, :, 0:1] + 1.0
    return lax.rsqrt(deg)


def _scale_body(deg_ref, x_ref, y_ref):
    dis = _dis_from(deg_ref)
    y_ref[...] = x_ref[...] * dis


def _layer1_body(deg_ref, s1_ref, x_ref, w1_ref, b1_ref, h_ref, y2_ref):
    dis = _dis_from(deg_ref)
    p = dis * (s1_ref[0] + s1_ref[1]) + (dis * dis) * x_ref[...]
    h = jnp.dot(p, w1_ref[...], preferred_element_type=jnp.float32)
    h = jnp.maximum(h + b1_ref[...], 0.0)
    h_ref[...] = h
    y2_ref[...] = h * dis


def _layer2_body(deg_ref, s2_ref, h_ref, w2_ref, b2_ref, out_ref):
    dis = _dis_from(deg_ref)
    p = dis * (s2_ref[0] + s2_ref[1]) + (dis * dis) * h_ref[...]
    out = jnp.dot(p, w2_ref[...], preferred_element_type=jnp.float32)
    out_ref[...] = out + b2_ref[...]


def _deg_spec():
    return pl.BlockSpec((2, _BM, DEG_W), lambda i: (0, i, 0))


def _row_spec(w=F):
    return pl.BlockSpec((_BM, w), lambda i: (i, 0))


def _part_spec(w=F):
    return pl.BlockSpec((2, _BM, w), lambda i: (0, i, 0))


def _full_spec(r, c):
    return pl.BlockSpec((r, c), lambda i: (0, 0))


_scale = pl.pallas_call(
    _scale_body,
    grid=(N // _BM,),
    in_specs=[_deg_spec(), _row_spec()],
    out_specs=_row_spec(),
    out_shape=jax.ShapeDtypeStruct((N, F), jnp.float32),
)

_layer1 = pl.pallas_call(
    _layer1_body,
    grid=(N // _BM,),
    in_specs=[_deg_spec(), _part_spec(), _row_spec(),
              _full_spec(F, F), _full_spec(1, F)],
    out_specs=[_row_spec(), _row_spec()],
    out_shape=[jax.ShapeDtypeStruct((N, F), jnp.float32),
               jax.ShapeDtypeStruct((N, F), jnp.float32)],
)

_layer2 = pl.pallas_call(
    _layer2_body,
    grid=(N // _BM,),
    in_specs=[_deg_spec(), _part_spec(), _row_spec(),
              _full_spec(F, 2 * F_OUT), _full_spec(1, 2 * F_OUT)],
    out_specs=_row_spec(2 * F_OUT),
    out_shape=jax.ShapeDtypeStruct((N, 2 * F_OUT), jnp.float32),
)


def kernel(x, edge_index, W1, b1, Wmu, bmu, Wsig, bsig):
    src3 = edge_index[0].reshape(NW, NCHUNK, CHUNK)
    dst3 = edge_index[1].reshape(NW, NCHUNK, CHUNK)
    z128 = jnp.zeros((ROWS_PT, F), jnp.float32)
    z16 = jnp.zeros((ROWS_PT, DEG_W), jnp.float32)
    ones16 = jnp.ones((CHUNK, DEG_W), jnp.float32)

    degp = _deg_kernel(dst3, z16, ones16)
    y1 = _scale(degp, x)
    s1 = _agg_kernel(y1, src3, dst3, z128)
    h, y2 = _layer1(degp, s1, x, W1, b1.reshape(1, F))
    s2 = _agg_kernel(y2, src3, dst3, z128)
    w2 = jnp.concatenate([Wmu, Wsig], axis=1)
    b2 = jnp.concatenate([bmu, bsig]).reshape(1, 2 * F_OUT)
    out = _layer2(degp, s2, h, w2, b2)
    return out[:, :F_OUT], out[:, F_OUT:]


# speedup vs baseline: 25.4365x; 25.4365x over previous
import functools

import jax
import jax.numpy as jnp
from jax import lax
from jax.experimental import pallas as pl
from jax.experimental.pallas import tpu as pltpu
from jax.experimental.pallas import tpu_sc as plsc

N = 10000
F = 128
F_OUT = 64
E = 320000
NC, NS = 2, 16
NW = NC * NS
EPW = E // NW
CHUNK = 80
NCHUNK = EPW // CHUNK
N_PAD = 10240
ROWS_PT = N_PAD // NS


def _mesh():
    return plsc.VectorSubcoreMesh(core_axis_name="c", subcore_axis_name="s")


@functools.partial(
    pl.kernel,
    out_type=jax.ShapeDtypeStruct((NC, N_PAD, F), jnp.float32),
    mesh=_mesh(),
    scratch_types=[
        pltpu.VMEM_SHARED((N_PAD, F), jnp.float32),
        pltpu.VMEM((2, 2, CHUNK), jnp.int32),
        pltpu.VMEM((CHUNK, F), jnp.float32),
        pltpu.SemaphoreType.DMA,
        pltpu.SemaphoreType.DMA,
    ],
)
def _deg_kernel(e_hbm, z128_hbm, ones_hbm, out_hbm, acc, db, ones, sem0, sem1):
    c = lax.axis_index("c")
    s = lax.axis_index("s")
    wid = s * NC + c
    sem = (sem0, sem1)
    pltpu.sync_copy(z128_hbm, acc.at[pl.ds(s * ROWS_PT, ROWS_PT)])
    pltpu.sync_copy(ones_hbm, ones)
    plsc.subcore_barrier()

    pltpu.async_copy(e_hbm.at[wid, 0], db.at[0], sem0)
    pltpu.async_copy(e_hbm.at[wid, 1], db.at[1], sem1)

    def step(j, b, do_pref):
        pltpu.make_async_copy(e_hbm.at[wid, j], db.at[b], sem[b]).wait()
        pltpu.sync_copy(ones, acc.at[db.at[b, 1]], add=True)
        if do_pref:
            pltpu.async_copy(e_hbm.at[wid, j + 2], db.at[b], sem[b])

    def body(i, carry):
        jo = 2 * i
        step(jo, 0, True)
        step(jo + 1, 1, True)
        return carry

    lax.fori_loop(0, (NCHUNK - 3) // 2, body, 0)
    step(NCHUNK - 3, 0, True)
    step(NCHUNK - 2, 1, False)
    step(NCHUNK - 1, 0, False)
    plsc.subcore_barrier()
    pltpu.sync_copy(acc.at[pl.ds(s * ROWS_PT, ROWS_PT)],
                    out_hbm.at[c, pl.ds(s * ROWS_PT, ROWS_PT)])


@functools.partial(
    pl.kernel,
    out_type=jax.ShapeDtypeStruct((NC, N_PAD, F), jnp.float32),
    mesh=_mesh(),
    scratch_types=[
        pltpu.VMEM_SHARED((N_PAD, F), jnp.float32),
        pltpu.VMEM((2, 2, CHUNK), jnp.int32),
        pltpu.VMEM((CHUNK, F), jnp.float32),
        pltpu.VMEM((CHUNK, F), jnp.float32),
        pltpu.SemaphoreType.DMA,
        pltpu.SemaphoreType.DMA,
        pltpu.SemaphoreType.DMA,
        pltpu.SemaphoreType.DMA,
    ],
)
def _agg_kernel(y_hbm, e_hbm, z128_hbm, out_hbm,
                acc, eb, rows0, rows1, semi0, semi1, semg0, semg1):
    c = lax.axis_index("c")
    s = lax.axis_index("s")
    wid = s * NC + c
    rows = (rows0, rows1)
    semi = (semi0, semi1)
    semg = (semg0, semg1)
    pltpu.sync_copy(z128_hbm, acc.at[pl.ds(s * ROWS_PT, ROWS_PT)])
    pltpu.sync_copy(e_hbm.at[wid, 0], eb.at[0])
    plsc.subcore_barrier()

    pltpu.async_copy(y_hbm.at[eb.at[0, 0]], rows0, semg0)
    pltpu.async_copy(e_hbm.at[wid, 1], eb.at[1], semi1)

    def step(j, b, bn, do_next, do_pref):
        if do_next:
            pltpu.make_async_copy(e_hbm.at[wid, j + 1], eb.at[bn],
                                  semi[bn]).wait()
            pltpu.async_copy(y_hbm.at[eb.at[bn, 0]], rows[bn], semg[bn])
        pltpu.make_async_copy(y_hbm.at[eb.at[b, 0]], rows[b], semg[b]).wait()
        pltpu.sync_copy(rows[b], acc.at[eb.at[b, 1]], add=True)
        if do_pref:
            pltpu.async_copy(e_hbm.at[wid, j + 2], eb.at[b], semi[b])

    def body(i, carry):
        jo = 2 * i
        step(jo, 0, 1, True, True)
        step(jo + 1, 1, 0, True, True)
        return carry

    lax.fori_loop(0, (NCHUNK - 3) // 2, body, 0)
    step(NCHUNK - 3, 0, 1, True, True)
    step(NCHUNK - 2, 1, 0, True, False)
    step(NCHUNK - 1, 0, 1, False, False)
    plsc.subcore_barrier()
    pltpu.sync_copy(acc.at[pl.ds(s * ROWS_PT, ROWS_PT)],
                    out_hbm.at[c, pl.ds(s * ROWS_PT, ROWS_PT)])


_BM = 1000


def _dis_from(deg_ref):
    deg = deg_ref[0, :, 0:1] + deg_ref[1, :, 0:1] + 1.0
    return lax.rsqrt(deg)


def _scale_body(deg_ref, x_ref, y_ref):
    dis = _dis_from(deg_ref)
    y_ref[...] = x_ref[...] * dis


def _layer1_body(deg_ref, s1_ref, x_ref, w1_ref, b1_ref, h_ref, y2_ref):
    dis = _dis_from(deg_ref)
    p = dis * (s1_ref[0] + s1_ref[1]) + (dis * dis) * x_ref[...]
    h = jnp.dot(p, w1_ref[...], preferred_element_type=jnp.float32)
    h = jnp.maximum(h + b1_ref[...], 0.0)
    h_ref[...] = h
    y2_ref[...] = h * dis


def _layer2_body(deg_ref, s2_ref, h_ref, w2_ref, b2_ref, out_ref):
    dis = _dis_from(deg_ref)
    p = dis * (s2_ref[0] + s2_ref[1]) + (dis * dis) * h_ref[...]
    out = jnp.dot(p, w2_ref[...], preferred_element_type=jnp.float32)
    out_ref[...] = out + b2_ref[...]


def _deg_spec():
    return pl.BlockSpec((2, _BM, F), lambda i: (0, i, 0))


def _row_spec(w=F):
    return pl.BlockSpec((_BM, w), lambda i: (i, 0))


def _part_spec(w=F):
    return pl.BlockSpec((2, _BM, w), lambda i: (0, i, 0))


def _full_spec(r, c):
    return pl.BlockSpec((r, c), lambda i: (0, 0))


_scale = pl.pallas_call(
    _scale_body,
    grid=(N // _BM,),
    in_specs=[_deg_spec(), _row_spec()],
    out_specs=_row_spec(),
    out_shape=jax.ShapeDtypeStruct((N, F), jnp.float32),
)

_layer1 = pl.pallas_call(
    _layer1_body,
    grid=(N // _BM,),
    in_specs=[_deg_spec(), _part_spec(), _row_spec(),
              _full_spec(F, F), _full_spec(1, F)],
    out_specs=[_row_spec(), _row_spec()],
    out_shape=[jax.ShapeDtypeStruct((N, F), jnp.float32),
               jax.ShapeDtypeStruct((N, F), jnp.float32)],
)

_layer2 = pl.pallas_call(
    _layer2_body,
    grid=(N // _BM,),
    in_specs=[_deg_spec(), _part_spec(), _row_spec(),
              _full_spec(F, 2 * F_OUT), _full_spec(1, 2 * F_OUT)],
    out_specs=_row_spec(2 * F_OUT),
    out_shape=jax.ShapeDtypeStruct((N, 2 * F_OUT), jnp.float32),
)


def kernel(x, edge_index, W1, b1, Wmu, bmu, Wsig, bsig):
    src3 = edge_index[0].reshape(NW, NCHUNK, CHUNK)
    dst3 = edge_index[1].reshape(NW, NCHUNK, CHUNK)
    z128 = jnp.zeros((ROWS_PT, F), jnp.float32)

    e3 = jnp.stack([src3, dst3], axis=2)

    ones128 = jnp.ones((CHUNK, F), jnp.float32)
    degp = _deg_kernel(e3, z128, ones128)
    y1 = _scale(degp, x)
    s1 = _agg_kernel(y1, e3, z128)
    h, y2 = _layer1(degp, s1, x, W1, b1.reshape(1, F))
    s2 = _agg_kernel(y2, e3, z128)
    w2 = jnp.concatenate([Wmu, Wsig], axis=1)
    b2 = jnp.concatenate([bmu, bsig]).reshape(1, 2 * F_OUT)
    out = _layer2(degp, s2, h, w2, b2)
    return out[:, :F_OUT], out[:, F_OUT:]


# speedup vs baseline: 32.3711x; 1.2726x over previous
import functools

import jax
import jax.numpy as jnp
from jax import lax
from jax.experimental import pallas as pl
from jax.experimental.pallas import tpu as pltpu
from jax.experimental.pallas import tpu_sc as plsc

N = 10000
F = 128
F_OUT = 64
E = 320000
NC, NS = 2, 16
NW = NC * NS
EPW = E // NW
CHUNK = 80
NCHUNK = EPW // CHUNK
N_PAD = 10240
ROWS_PT = N_PAD // NS


def _mesh():
    return plsc.VectorSubcoreMesh(core_axis_name="c", subcore_axis_name="s")


_NEB = 6
_NRW = 3


@functools.partial(
    pl.kernel,
    out_type=jax.ShapeDtypeStruct((NC, N_PAD, F), jnp.float32),
    mesh=_mesh(),
    scratch_types=[
        pltpu.VMEM_SHARED((N_PAD, F), jnp.float32),
        pltpu.VMEM((_NEB, 2, CHUNK), jnp.int32),
        pltpu.VMEM((CHUNK, F), jnp.float32),
    ] + [pltpu.SemaphoreType.DMA] * (_NEB + _NRW),
)
def _deg_kernel(e_hbm, z128_hbm, ones_hbm, out_hbm, acc, eb, ones,
                si0, si1, si2, si3, si4, si5, ss0, ss1, ss2):
    c = lax.axis_index("c")
    s = lax.axis_index("s")
    wid = s * NC + c
    semi = (si0, si1, si2, si3, si4, si5)
    sems = (ss0, ss1, ss2)

    for j in range(4):
        pltpu.async_copy(e_hbm.at[wid, j], eb.at[j], semi[j])
    pltpu.sync_copy(z128_hbm, acc.at[pl.ds(s * ROWS_PT, ROWS_PT)])
    pltpu.sync_copy(ones_hbm, ones)
    plsc.subcore_barrier()

    def step(j, m, wait_sc, do_pref):
        e, e4 = m % _NEB, (m + 4) % _NEB
        r, r1 = m % _NRW, (m + 1) % _NRW
        if wait_sc:
            pltpu.make_async_copy(ones, acc.at[eb.at[e4, 1]],
                                  sems[r1]).wait()
        if do_pref:
            pltpu.async_copy(e_hbm.at[wid, j + 4], eb.at[e4], semi[e4])
        pltpu.make_async_copy(e_hbm.at[wid, j], eb.at[e], semi[e]).wait()
        pltpu.make_async_copy(ones, acc.at[eb.at[e, 1]],
                              sems[r]).start(add=True)

    step(0, 0, False, True)
    step(1, 1, False, True)
    for m in range(2, 6):
        step(m, m, True, True)

    def body(i, carry):
        base = 6 * i
        for m in range(6):
            step(base + m, m, True, True)
        return carry

    lax.fori_loop(1, (NCHUNK - 5) // 6, body, 0)
    step(NCHUNK - 5, 0, True, True)
    for t, m in ((NCHUNK - 4, 1), (NCHUNK - 3, 2), (NCHUNK - 2, 3),
                 (NCHUNK - 1, 4)):
        step(t, m, True, False)
    pltpu.make_async_copy(ones, acc.at[eb.at[3, 1]], sems[0]).wait()
    pltpu.make_async_copy(ones, acc.at[eb.at[4, 1]], sems[1]).wait()
    plsc.subcore_barrier()
    pltpu.sync_copy(acc.at[pl.ds(s * ROWS_PT, ROWS_PT)],
                    out_hbm.at[c, pl.ds(s * ROWS_PT, ROWS_PT)])


@functools.partial(
    pl.kernel,
    out_type=jax.ShapeDtypeStruct((NC, N_PAD, F), jnp.float32),
    mesh=_mesh(),
    scratch_types=[
        pltpu.VMEM_SHARED((N_PAD, F), jnp.float32),
        pltpu.VMEM((_NEB, 2, CHUNK), jnp.int32),
        pltpu.VMEM((CHUNK, F), jnp.float32),
        pltpu.VMEM((CHUNK, F), jnp.float32),
        pltpu.VMEM((CHUNK, F), jnp.float32),
    ] + [pltpu.SemaphoreType.DMA] * (_NEB + 2 * _NRW),
)
def _agg_kernel(y_hbm, e_hbm, z128_hbm, out_hbm, acc, eb,
                rows0, rows1, rows2,
                si0, si1, si2, si3, si4, si5,
                sg0, sg1, sg2, ss0, ss1, ss2):
    c = lax.axis_index("c")
    s = lax.axis_index("s")
    wid = s * NC + c
    rows = (rows0, rows1, rows2)
    semi = (si0, si1, si2, si3, si4, si5)
    semg = (sg0, sg1, sg2)
    sems = (ss0, ss1, ss2)

    for j in range(4):
        pltpu.async_copy(e_hbm.at[wid, j], eb.at[j], semi[j])
    pltpu.sync_copy(z128_hbm, acc.at[pl.ds(s * ROWS_PT, ROWS_PT)])
    pltpu.make_async_copy(e_hbm.at[wid, 0], eb.at[0], semi[0]).wait()
    pltpu.async_copy(y_hbm.at[eb.at[0, 0]], rows0, semg[0])
    plsc.subcore_barrier()

    def step(j, m, wait_sc, do_pref, do_g):
        e, e1, e4 = m % _NEB, (m + 1) % _NEB, (m + 4) % _NEB
        r, r1 = m % _NRW, (m + 1) % _NRW
        if wait_sc:
            pltpu.make_async_copy(rows[r1], acc.at[eb.at[e4, 1]],
                                  sems[r1]).wait()
        if do_pref:
            pltpu.async_copy(e_hbm.at[wid, j + 4], eb.at[e4], semi[e4])
        if do_g:
            pltpu.make_async_copy(e_hbm.at[wid, j + 1], eb.at[e1],
                                  semi[e1]).wait()
            pltpu.async_copy(y_hbm.at[eb.at[e1, 0]], rows[r1], semg[r1])
        pltpu.make_async_copy(y_hbm.at[eb.at[e, 0]], rows[r], semg[r]).wait()
        pltpu.make_async_copy(rows[r], acc.at[eb.at[e, 1]],
                              sems[r]).start(add=True)

    step(0, 0, False, True, True)
    step(1, 1, False, True, True)
    for m in range(2, 6):
        step(m, m, True, True, True)

    def body(i, carry):
        base = 6 * i
        for m in range(6):
            step(base + m, m, True, True, True)
        return carry

    lax.fori_loop(1, (NCHUNK - 5) // 6, body, 0)
    step(NCHUNK - 5, 0, True, True, True)
    step(NCHUNK - 4, 1, True, False, True)
    step(NCHUNK - 3, 2, True, False, True)
    step(NCHUNK - 2, 3, True, False, True)
    step(NCHUNK - 1, 4, True, False, False)
    pltpu.make_async_copy(rows[0], acc.at[eb.at[3, 1]], sems[0]).wait()
    pltpu.make_async_copy(rows[1], acc.at[eb.at[4, 1]], sems[1]).wait()
    plsc.subcore_barrier()
    pltpu.sync_copy(acc.at[pl.ds(s * ROWS_PT, ROWS_PT)],
                    out_hbm.at[c, pl.ds(s * ROWS_PT, ROWS_PT)])


_BM = 1000


def _dis_from(deg_ref):
    deg = deg_ref[0, :, 0:1] + deg_ref[1, :, 0:1] + 1.0
    return lax.rsqrt(deg)


def _scale_body(deg_ref, x_ref, y_ref):
    dis = _dis_from(deg_ref)
    y_ref[...] = x_ref[...] * dis


def _layer1_body(deg_ref, s1_ref, x_ref, w1_ref, b1_ref, h_ref, y2_ref):
    dis = _dis_from(deg_ref)
    p = dis * (s1_ref[0] + s1_ref[1]) + (dis * dis) * x_ref[...]
    h = jnp.dot(p, w1_ref[...], preferred_element_type=jnp.float32)
    h = jnp.maximum(h + b1_ref[...], 0.0)
    h_ref[...] = h
    y2_ref[...] = h * dis


def _layer2_body(deg_ref, s2_ref, h_ref, w2_ref, b2_ref, out_ref):
    dis = _dis_from(deg_ref)
    p = dis * (s2_ref[0] + s2_ref[1]) + (dis * dis) * h_ref[...]
    out = jnp.dot(p, w2_ref[...], preferred_element_type=jnp.float32)
    out_ref[...] = out + b2_ref[...]


def _deg_spec():
    return pl.BlockSpec((2, _BM, F), lambda i: (0, i, 0))


def _row_spec(w=F):
    return pl.BlockSpec((_BM, w), lambda i: (i, 0))


def _part_spec(w=F):
    return pl.BlockSpec((2, _BM, w), lambda i: (0, i, 0))


def _full_spec(r, c):
    return pl.BlockSpec((r, c), lambda i: (0, 0))


_scale = pl.pallas_call(
    _scale_body,
    grid=(N // _BM,),
    in_specs=[_deg_spec(), _row_spec()],
    out_specs=_row_spec(),
    out_shape=jax.ShapeDtypeStruct((N, F), jnp.float32),
)

_layer1 = pl.pallas_call(
    _layer1_body,
    grid=(N // _BM,),
    in_specs=[_deg_spec(), _part_spec(), _row_spec(),
              _full_spec(F, F), _full_spec(1, F)],
    out_specs=[_row_spec(), _row_spec()],
    out_shape=[jax.ShapeDtypeStruct((N, F), jnp.float32),
               jax.ShapeDtypeStruct((N, F), jnp.float32)],
)

_layer2 = pl.pallas_call(
    _layer2_body,
    grid=(N // _BM,),
    in_specs=[_deg_spec(), _part_spec(), _row_spec(),
              _full_spec(F, 2 * F_OUT), _full_spec(1, 2 * F_OUT)],
    out_specs=_row_spec(2 * F_OUT),
    out_shape=jax.ShapeDtypeStruct((N, 2 * F_OUT), jnp.float32),
)


def kernel(x, edge_index, W1, b1, Wmu, bmu, Wsig, bsig):
    src3 = edge_index[0].reshape(NW, NCHUNK, CHUNK)
    dst3 = edge_index[1].reshape(NW, NCHUNK, CHUNK)
    z128 = jnp.zeros((ROWS_PT, F), jnp.float32)

    e3 = jnp.stack([src3, dst3], axis=2)

    ones128 = jnp.ones((CHUNK, F), jnp.float32)
    degp = _deg_kernel(e3, z128, ones128)
    y1 = _scale(degp, x)
    s1 = _agg_kernel(y1, e3, z128)
    h, y2 = _layer1(degp, s1, x, W1, b1.reshape(1, F))
    s2 = _agg_kernel(y2, e3, z128)
    w2 = jnp.concatenate([Wmu, Wsig], axis=1)
    b2 = jnp.concatenate([bmu, bsig]).reshape(1, 2 * F_OUT)
    out = _layer2(degp, s2, h, w2, b2)
    return out[:, :F_OUT], out[:, F_OUT:]


# speedup vs baseline: 32.7628x; 1.0121x over previous
import functools

import jax
import jax.numpy as jnp
from jax import lax
from jax.experimental import pallas as pl
from jax.experimental.pallas import tpu as pltpu
from jax.experimental.pallas import tpu_sc as plsc

N = 10000
F = 128
F_OUT = 64
E = 320000
NC, NS = 2, 16
NW = NC * NS
EPW = E // NW
CHUNK = 80
NCHUNK = EPW // CHUNK
N_PAD = 10240
ROWS_PT = N_PAD // NS


def _mesh():
    return plsc.VectorSubcoreMesh(core_axis_name="c", subcore_axis_name="s")


_NEB = 6
_NRW = 3


@functools.partial(
    pl.kernel,
    out_type=jax.ShapeDtypeStruct((NC, N_PAD, F), jnp.float32),
    mesh=_mesh(),
    scratch_types=[
        pltpu.VMEM_SHARED((N_PAD, F), jnp.float32),
        pltpu.VMEM((_NEB, 2, CHUNK), jnp.int32),
        pltpu.VMEM((CHUNK, F), jnp.float32),
    ] + [pltpu.SemaphoreType.DMA] * (_NEB + _NRW),
)
def _deg_kernel(e_hbm, z128_hbm, ones_hbm, out_hbm, acc, eb, ones,
                si0, si1, si2, si3, si4, si5, ss0, ss1, ss2):
    c = lax.axis_index("c")
    s = lax.axis_index("s")
    wid = s * NC + c
    semi = (si0, si1, si2, si3, si4, si5)
    sems = (ss0, ss1, ss2)

    for j in range(4):
        pltpu.async_copy(e_hbm.at[wid, j], eb.at[j], semi[j])
    pltpu.sync_copy(z128_hbm, acc.at[pl.ds(s * ROWS_PT, ROWS_PT)])
    pltpu.sync_copy(ones_hbm, ones)
    plsc.subcore_barrier()

    def step(j, m, wait_sc, do_pref):
        e, e4 = m % _NEB, (m + 4) % _NEB
        r, r1 = m % _NRW, (m + 1) % _NRW
        if wait_sc:
            pltpu.make_async_copy(ones, acc.at[eb.at[e4, 1]],
                                  sems[r1]).wait()
        if do_pref:
            pltpu.async_copy(e_hbm.at[wid, j + 4], eb.at[e4], semi[e4])
        pltpu.make_async_copy(e_hbm.at[wid, j], eb.at[e], semi[e]).wait()
        pltpu.make_async_copy(ones, acc.at[eb.at[e, 1]],
                              sems[r]).start(add=True)

    step(0, 0, False, True)
    step(1, 1, False, True)
    for m in range(2, 6):
        step(m, m, True, True)

    def body(i, carry):
        base = 6 * i
        for m in range(6):
            step(base + m, m, True, True)
        return carry

    lax.fori_loop(1, (NCHUNK - 5) // 6, body, 0)
    step(NCHUNK - 5, 0, True, True)
    for t, m in ((NCHUNK - 4, 1), (NCHUNK - 3, 2), (NCHUNK - 2, 3),
                 (NCHUNK - 1, 4)):
        step(t, m, True, False)
    pltpu.make_async_copy(ones, acc.at[eb.at[3, 1]], sems[0]).wait()
    pltpu.make_async_copy(ones, acc.at[eb.at[4, 1]], sems[1]).wait()
    plsc.subcore_barrier()
    pltpu.sync_copy(acc.at[pl.ds(s * ROWS_PT, ROWS_PT)],
                    out_hbm.at[c, pl.ds(s * ROWS_PT, ROWS_PT)])


@functools.partial(
    pl.kernel,
    out_type=jax.ShapeDtypeStruct((NC, N_PAD, F), jnp.float32),
    mesh=_mesh(),
    scratch_types=[
        pltpu.VMEM_SHARED((N_PAD, F), jnp.float32),
        pltpu.VMEM((_NEB, 2, CHUNK), jnp.int32),
        pltpu.VMEM((CHUNK, F), jnp.float32),
        pltpu.VMEM((CHUNK, F), jnp.float32),
        pltpu.VMEM((CHUNK, F), jnp.float32),
    ] + [pltpu.SemaphoreType.DMA] * (_NEB + 2 * _NRW),
)
def _agg_kernel(y_hbm, e_hbm, z128_hbm, out_hbm, acc, eb,
                rows0, rows1, rows2,
                si0, si1, si2, si3, si4, si5,
                sg0, sg1, sg2, ss0, ss1, ss2):
    c = lax.axis_index("c")
    s = lax.axis_index("s")
    wid = s * NC + c
    rows = (rows0, rows1, rows2)
    semi = (si0, si1, si2, si3, si4, si5)
    semg = (sg0, sg1, sg2)
    sems = (ss0, ss1, ss2)

    for j in range(4):
        pltpu.async_copy(e_hbm.at[wid, j], eb.at[j], semi[j])
    pltpu.sync_copy(z128_hbm, acc.at[pl.ds(s * ROWS_PT, ROWS_PT)])
    pltpu.make_async_copy(e_hbm.at[wid, 0], eb.at[0], semi[0]).wait()
    pltpu.async_copy(y_hbm.at[eb.at[0, 0]], rows0, semg[0])
    plsc.subcore_barrier()

    def step(j, m, wait_sc, do_pref, do_g):
        e, e1, e4 = m % _NEB, (m + 1) % _NEB, (m + 4) % _NEB
        r, r1 = m % _NRW, (m + 1) % _NRW
        if wait_sc:
            pltpu.make_async_copy(rows[r1], acc.at[eb.at[e4, 1]],
                                  sems[r1]).wait()
        if do_pref:
            pltpu.async_copy(e_hbm.at[wid, j + 4], eb.at[e4], semi[e4])
        if do_g:
            pltpu.make_async_copy(e_hbm.at[wid, j + 1], eb.at[e1],
                                  semi[e1]).wait()
            pltpu.async_copy(y_hbm.at[eb.at[e1, 0]], rows[r1], semg[r1])
        pltpu.make_async_copy(y_hbm.at[eb.at[e, 0]], rows[r], semg[r]).wait()
        pltpu.make_async_copy(rows[r], acc.at[eb.at[e, 1]],
                              sems[r]).start(add=True)

    step(0, 0, False, True, True)
    step(1, 1, False, True, True)
    for m in range(2, 6):
        step(m, m, True, True, True)

    def body(i, carry):
        base = 6 * i
        for m in range(6):
            step(base + m, m, True, True, True)
        return carry

    lax.fori_loop(1, (NCHUNK - 5) // 6, body, 0)
    step(NCHUNK - 5, 0, True, True, True)
    step(NCHUNK - 4, 1, True, False, True)
    step(NCHUNK - 3, 2, True, False, True)
    step(NCHUNK - 2, 3, True, False, True)
    step(NCHUNK - 1, 4, True, False, False)
    pltpu.make_async_copy(rows[0], acc.at[eb.at[3, 1]], sems[0]).wait()
    pltpu.make_async_copy(rows[1], acc.at[eb.at[4, 1]], sems[1]).wait()
    plsc.subcore_barrier()
    pltpu.sync_copy(acc.at[pl.ds(s * ROWS_PT, ROWS_PT)],
                    out_hbm.at[c, pl.ds(s * ROWS_PT, ROWS_PT)])


_BM = 1000


def _dis_from(deg_ref):
    deg = deg_ref[0, :, 0:1] + deg_ref[1, :, 0:1] + 1.0
    return lax.rsqrt(deg)


def _scale_body(deg_ref, x_ref, y_ref):
    dis = _dis_from(deg_ref)
    y_ref[...] = x_ref[...] * dis


def _layer1_body(deg_ref, s1_ref, x_ref, w1_ref, b1_ref, h_ref, y2_ref):
    dis = _dis_from(deg_ref)
    p = dis * (s1_ref[0] + s1_ref[1]) + (dis * dis) * x_ref[...]
    h = jnp.dot(p, w1_ref[...], preferred_element_type=jnp.float32)
    h = jnp.maximum(h + b1_ref[...], 0.0)
    h_ref[...] = h
    y2_ref[...] = h * dis


def _layer2_body(deg_ref, s2_ref, h_ref, w2_ref, b2_ref, mu_ref, sig_ref):
    dis = _dis_from(deg_ref)
    p = dis * (s2_ref[0] + s2_ref[1]) + (dis * dis) * h_ref[...]
    out = jnp.dot(p, w2_ref[...], preferred_element_type=jnp.float32)
    out = out + b2_ref[...]
    mu_ref[...] = out[:, :F_OUT]
    sig_ref[...] = out[:, F_OUT:]


def _deg_spec():
    return pl.BlockSpec((2, _BM, F), lambda i: (0, i, 0))


def _row_spec(w=F):
    return pl.BlockSpec((_BM, w), lambda i: (i, 0))


def _part_spec(w=F):
    return pl.BlockSpec((2, _BM, w), lambda i: (0, i, 0))


def _full_spec(r, c):
    return pl.BlockSpec((r, c), lambda i: (0, 0))


_scale = pl.pallas_call(
    _scale_body,
    grid=(N // _BM,),
    in_specs=[_deg_spec(), _row_spec()],
    out_specs=_row_spec(),
    out_shape=jax.ShapeDtypeStruct((N, F), jnp.float32),
)

_layer1 = pl.pallas_call(
    _layer1_body,
    grid=(N // _BM,),
    in_specs=[_deg_spec(), _part_spec(), _row_spec(),
              _full_spec(F, F), _full_spec(1, F)],
    out_specs=[_row_spec(), _row_spec()],
    out_shape=[jax.ShapeDtypeStruct((N, F), jnp.float32),
               jax.ShapeDtypeStruct((N, F), jnp.float32)],
)

_layer2 = pl.pallas_call(
    _layer2_body,
    grid=(N // _BM,),
    in_specs=[_deg_spec(), _part_spec(), _row_spec(),
              _full_spec(F, 2 * F_OUT), _full_spec(1, 2 * F_OUT)],
    out_specs=[_row_spec(F_OUT), _row_spec(F_OUT)],
    out_shape=[jax.ShapeDtypeStruct((N, F_OUT), jnp.float32),
               jax.ShapeDtypeStruct((N, F_OUT), jnp.float32)],
)


def kernel(x, edge_index, W1, b1, Wmu, bmu, Wsig, bsig):
    src3 = edge_index[0].reshape(NW, NCHUNK, CHUNK)
    dst3 = edge_index[1].reshape(NW, NCHUNK, CHUNK)
    z128 = jnp.zeros((ROWS_PT, F), jnp.float32)

    e3 = jnp.stack([src3, dst3], axis=2)

    ones128 = jnp.ones((CHUNK, F), jnp.float32)
    degp = _deg_kernel(e3, z128, ones128)
    y1 = _scale(degp, x)
    s1 = _agg_kernel(y1, e3, z128)
    h, y2 = _layer1(degp, s1, x, W1, b1.reshape(1, F))
    s2 = _agg_kernel(y2, e3, z128)
    w2 = jnp.concatenate([Wmu, Wsig], axis=1)
    b2 = jnp.concatenate([bmu, bsig]).reshape(1, 2 * F_OUT)
    mu, sig = _layer2(degp, s2, h, w2, b2)
    return mu, sig


# speedup vs baseline: 32.7804x; 1.0005x over previous
import functools

import jax
import jax.numpy as jnp
from jax import lax
from jax.experimental import pallas as pl
from jax.experimental.pallas import tpu as pltpu
from jax.experimental.pallas import tpu_sc as plsc

N = 10000
F = 128
F_OUT = 64
E = 320000
NC, NS = 2, 16
NW = NC * NS
EPW = E // NW
CHUNK = 80
NCHUNK = EPW // CHUNK
N_PAD = 10240
ROWS_PT = N_PAD // NS


def _mesh():
    return plsc.VectorSubcoreMesh(core_axis_name="c", subcore_axis_name="s")


_NEB = 6
_NRW = 3


@functools.partial(
    pl.kernel,
    out_type=jax.ShapeDtypeStruct((NC, N_PAD, F), jnp.float32),
    mesh=_mesh(),
    scratch_types=[
        pltpu.VMEM_SHARED((N_PAD, F), jnp.float32),
        pltpu.VMEM((_NEB, 2, CHUNK), jnp.int32),
        pltpu.VMEM((CHUNK, F), jnp.float32),
    ] + [pltpu.SemaphoreType.DMA] * (_NEB + _NRW),
)
def _deg_kernel(e_hbm, z128_hbm, ones_hbm, out_hbm, acc, eb, ones,
                si0, si1, si2, si3, si4, si5, ss0, ss1, ss2):
    c = lax.axis_index("c")
    s = lax.axis_index("s")
    wid = s * NC + c
    semi = (si0, si1, si2, si3, si4, si5)
    sems = (ss0, ss1, ss2)

    for j in range(4):
        pltpu.async_copy(e_hbm.at[wid, j], eb.at[j], semi[j])
    pltpu.sync_copy(z128_hbm, acc.at[pl.ds(s * ROWS_PT, ROWS_PT)])
    pltpu.sync_copy(ones_hbm, ones)
    plsc.subcore_barrier()

    def step(j, m, wait_sc, do_pref):
        e, e4 = m % _NEB, (m + 4) % _NEB
        r, r1 = m % _NRW, (m + 1) % _NRW
        if wait_sc:
            pltpu.make_async_copy(ones, acc.at[eb.at[e4, 1]],
                                  sems[r1]).wait()
        if do_pref:
            pltpu.async_copy(e_hbm.at[wid, j + 4], eb.at[e4], semi[e4])
        pltpu.make_async_copy(e_hbm.at[wid, j], eb.at[e], semi[e]).wait()
        pltpu.make_async_copy(ones, acc.at[eb.at[e, 1]],
                              sems[r]).start(add=True)

    step(0, 0, False, True)
    step(1, 1, False, True)
    for m in range(2, 6):
        step(m, m, True, True)

    def body(i, carry):
        base = 6 * i
        for m in range(6):
            step(base + m, m, True, True)
        return carry

    lax.fori_loop(1, (NCHUNK - 5) // 6, body, 0)
    step(NCHUNK - 5, 0, True, True)
    for t, m in ((NCHUNK - 4, 1), (NCHUNK - 3, 2), (NCHUNK - 2, 3),
                 (NCHUNK - 1, 4)):
        step(t, m, True, False)
    pltpu.make_async_copy(ones, acc.at[eb.at[3, 1]], sems[0]).wait()
    pltpu.make_async_copy(ones, acc.at[eb.at[4, 1]], sems[1]).wait()
    plsc.subcore_barrier()
    pltpu.sync_copy(acc.at[pl.ds(s * ROWS_PT, ROWS_PT)],
                    out_hbm.at[c, pl.ds(s * ROWS_PT, ROWS_PT)])


@functools.partial(
    pl.kernel,
    out_type=jax.ShapeDtypeStruct((NC, N_PAD, F), jnp.float32),
    mesh=_mesh(),
    scratch_types=[
        pltpu.VMEM_SHARED((N_PAD, F), jnp.float32),
        pltpu.VMEM((_NEB, 2, CHUNK), jnp.int32),
        pltpu.VMEM((CHUNK, F), jnp.float32),
        pltpu.VMEM((CHUNK, F), jnp.float32),
        pltpu.VMEM((CHUNK, F), jnp.float32),
    ] + [pltpu.SemaphoreType.DMA] * (_NEB + 2 * _NRW),
)
def _agg_kernel(y_hbm, e_hbm, z128_hbm, out_hbm, acc, eb,
                rows0, rows1, rows2,
                si0, si1, si2, si3, si4, si5,
                sg0, sg1, sg2, ss0, ss1, ss2):
    c = lax.axis_index("c")
    s = lax.axis_index("s")
    wid = s * NC + c
    rows = (rows0, rows1, rows2)
    semi = (si0, si1, si2, si3, si4, si5)
    semg = (sg0, sg1, sg2)
    sems = (ss0, ss1, ss2)

    for j in range(4):
        pltpu.async_copy(e_hbm.at[wid, j], eb.at[j], semi[j])
    pltpu.sync_copy(z128_hbm, acc.at[pl.ds(s * ROWS_PT, ROWS_PT)])
    pltpu.make_async_copy(e_hbm.at[wid, 0], eb.at[0], semi[0]).wait()
    pltpu.async_copy(y_hbm.at[eb.at[0, 0]], rows0, semg[0])
    plsc.subcore_barrier()

    def step(j, m, wait_sc, do_pref, do_g):
        e, e1, e4 = m % _NEB, (m + 1) % _NEB, (m + 4) % _NEB
        r, r1 = m % _NRW, (m + 1) % _NRW
        if wait_sc:
            pltpu.make_async_copy(rows[r1], acc.at[eb.at[e4, 1]],
                                  sems[r1]).wait()
        if do_pref:
            pltpu.async_copy(e_hbm.at[wid, j + 4], eb.at[e4], semi[e4])
        if do_g:
            pltpu.make_async_copy(e_hbm.at[wid, j + 1], eb.at[e1],
                                  semi[e1]).wait()
            pltpu.async_copy(y_hbm.at[eb.at[e1, 0]], rows[r1], semg[r1])
        pltpu.make_async_copy(y_hbm.at[eb.at[e, 0]], rows[r], semg[r]).wait()
        pltpu.make_async_copy(rows[r], acc.at[eb.at[e, 1]],
                              sems[r]).start(add=True)

    step(0, 0, False, True, True)
    step(1, 1, False, True, True)
    for m in range(2, 6):
        step(m, m, True, True, True)

    def body(i, carry):
        base = 6 * i
        for m in range(6):
            step(base + m, m, True, True, True)
        return carry

    lax.fori_loop(1, (NCHUNK - 5) // 6, body, 0)
    step(NCHUNK - 5, 0, True, True, True)
    step(NCHUNK - 4, 1, True, False, True)
    step(NCHUNK - 3, 2, True, False, True)
    step(NCHUNK - 2, 3, True, False, True)
    step(NCHUNK - 1, 4, True, False, False)
    pltpu.make_async_copy(rows[0], acc.at[eb.at[3, 1]], sems[0]).wait()
    pltpu.make_async_copy(rows[1], acc.at[eb.at[4, 1]], sems[1]).wait()
    plsc.subcore_barrier()
    pltpu.sync_copy(acc.at[pl.ds(s * ROWS_PT, ROWS_PT)],
                    out_hbm.at[c, pl.ds(s * ROWS_PT, ROWS_PT)])


_BM = 1000


def _dis_from(deg_ref):
    deg = deg_ref[0, :, 0:1] + deg_ref[1, :, 0:1] + 1.0
    return lax.rsqrt(deg)


def _xw_body(x_ref, w1_ref, xw_ref):
    xw_ref[...] = jnp.dot(x_ref[...], w1_ref[...],
                          preferred_element_type=jnp.float32)


def _scale_body(deg_ref, x_ref, y_ref):
    dis = _dis_from(deg_ref)
    y_ref[...] = x_ref[...] * dis


def _layer1_body(deg_ref, s1_ref, xw_ref, b1_ref, w2_ref, h_ref, y2_ref):
    dis = _dis_from(deg_ref)
    p = dis * (s1_ref[0] + s1_ref[1]) + (dis * dis) * xw_ref[...]
    h = jnp.maximum(p + b1_ref[...], 0.0)
    hw = jnp.dot(h, w2_ref[...], preferred_element_type=jnp.float32)
    h_ref[...] = hw
    y2_ref[...] = hw * dis


def _layer2_body(deg_ref, s2_ref, hw_ref, b2_ref, mu_ref, sig_ref):
    dis = _dis_from(deg_ref)
    out = dis * (s2_ref[0] + s2_ref[1]) + (dis * dis) * hw_ref[...]
    out = out + b2_ref[...]
    mu_ref[...] = out[:, :F_OUT]
    sig_ref[...] = out[:, F_OUT:]


def _deg_spec():
    return pl.BlockSpec((2, _BM, F), lambda i: (0, i, 0))


def _row_spec(w=F):
    return pl.BlockSpec((_BM, w), lambda i: (i, 0))


def _part_spec(w=F):
    return pl.BlockSpec((2, _BM, w), lambda i: (0, i, 0))


def _full_spec(r, c):
    return pl.BlockSpec((r, c), lambda i: (0, 0))


_xw = pl.pallas_call(
    _xw_body,
    grid=(N // _BM,),
    in_specs=[_row_spec(), _full_spec(F, F)],
    out_specs=_row_spec(),
    out_shape=jax.ShapeDtypeStruct((N, F), jnp.float32),
)

_scale = pl.pallas_call(
    _scale_body,
    grid=(N // _BM,),
    in_specs=[_deg_spec(), _row_spec()],
    out_specs=_row_spec(),
    out_shape=jax.ShapeDtypeStruct((N, F), jnp.float32),
)

_layer1 = pl.pallas_call(
    _layer1_body,
    grid=(N // _BM,),
    in_specs=[_deg_spec(), _part_spec(), _row_spec(),
              _full_spec(1, F), _full_spec(F, 2 * F_OUT)],
    out_specs=[_row_spec(), _row_spec()],
    out_shape=[jax.ShapeDtypeStruct((N, F), jnp.float32),
               jax.ShapeDtypeStruct((N, F), jnp.float32)],
)

_layer2 = pl.pallas_call(
    _layer2_body,
    grid=(N // _BM,),
    in_specs=[_deg_spec(), _part_spec(), _row_spec(),
              _full_spec(1, 2 * F_OUT)],
    out_specs=[_row_spec(F_OUT), _row_spec(F_OUT)],
    out_shape=[jax.ShapeDtypeStruct((N, F_OUT), jnp.float32),
               jax.ShapeDtypeStruct((N, F_OUT), jnp.float32)],
)


def kernel(x, edge_index, W1, b1, Wmu, bmu, Wsig, bsig):
    src3 = edge_index[0].reshape(NW, NCHUNK, CHUNK)
    dst3 = edge_index[1].reshape(NW, NCHUNK, CHUNK)
    z128 = jnp.zeros((ROWS_PT, F), jnp.float32)

    e3 = jnp.stack([src3, dst3], axis=2)

    ones128 = jnp.ones((CHUNK, F), jnp.float32)
    w2 = jnp.concatenate([Wmu, Wsig], axis=1)
    b2 = jnp.concatenate([bmu, bsig]).reshape(1, 2 * F_OUT)
    xw = _xw(x, W1)
    degp = _deg_kernel(e3, z128, ones128)
    y1 = _scale(degp, xw)
    s1 = _agg_kernel(y1, e3, z128)
    hw, y2 = _layer1(degp, s1, xw, b1.reshape(1, F), w2)
    s2 = _agg_kernel(y2, e3, z128)
    mu, sig = _layer2(degp, s2, hw, b2)
    return mu, sig


# speedup vs baseline: 33.2467x; 1.0142x over previous
import functools

import jax
import jax.numpy as jnp
from jax import lax
from jax.experimental import pallas as pl
from jax.experimental.pallas import tpu as pltpu
from jax.experimental.pallas import tpu_sc as plsc

N = 10000
F = 128
F_OUT = 64
E = 320000
NC, NS = 2, 16
NW = NC * NS
EPW = E // NW
CHUNK = 96
NCHUNK = 107
PAD_E = NCHUNK * CHUNK - EPW
N_PAD = 10240
ROWS_PT = N_PAD // NS


def _mesh():
    return plsc.VectorSubcoreMesh(core_axis_name="c", subcore_axis_name="s")


_NEB = 6
_NRW = 3


@functools.partial(
    pl.kernel,
    out_type=jax.ShapeDtypeStruct((NC, N_PAD, F), jnp.float32),
    mesh=_mesh(),
    scratch_types=[
        pltpu.VMEM_SHARED((N_PAD, F), jnp.float32),
        pltpu.VMEM((_NEB, 2, CHUNK), jnp.int32),
        pltpu.VMEM((CHUNK, F), jnp.float32),
    ] + [pltpu.SemaphoreType.DMA] * (_NEB + _NRW),
)
def _deg_kernel(e_hbm, z128_hbm, ones_hbm, out_hbm, acc, eb, ones,
                si0, si1, si2, si3, si4, si5, ss0, ss1, ss2):
    c = lax.axis_index("c")
    s = lax.axis_index("s")
    wid = s * NC + c
    semi = (si0, si1, si2, si3, si4, si5)
    sems = (ss0, ss1, ss2)

    for j in range(4):
        pltpu.async_copy(e_hbm.at[wid, j], eb.at[j], semi[j])
    pltpu.sync_copy(z128_hbm, acc.at[pl.ds(s * ROWS_PT, ROWS_PT)])
    pltpu.sync_copy(ones_hbm, ones)
    plsc.subcore_barrier()

    def step(j, m, wait_sc, do_pref):
        e, e4 = m % _NEB, (m + 4) % _NEB
        r, r1 = m % _NRW, (m + 1) % _NRW
        if wait_sc:
            pltpu.make_async_copy(ones, acc.at[eb.at[e4, 1]],
                                  sems[r1]).wait()
        if do_pref:
            pltpu.async_copy(e_hbm.at[wid, j + 4], eb.at[e4], semi[e4])
        pltpu.make_async_copy(e_hbm.at[wid, j], eb.at[e], semi[e]).wait()
        pltpu.make_async_copy(ones, acc.at[eb.at[e, 1]],
                              sems[r]).start(add=True)

    step(0, 0, False, True)
    step(1, 1, False, True)
    for m in range(2, 6):
        step(m, m, True, True)

    def body(i, carry):
        base = 6 * i
        for m in range(6):
            step(base + m, m, True, True)
        return carry

    lax.fori_loop(1, (NCHUNK - 5) // 6, body, 0)
    step(NCHUNK - 5, 0, True, True)
    for t, m in ((NCHUNK - 4, 1), (NCHUNK - 3, 2), (NCHUNK - 2, 3),
                 (NCHUNK - 1, 4)):
        step(t, m, True, False)
    pltpu.make_async_copy(ones, acc.at[eb.at[3, 1]], sems[0]).wait()
    pltpu.make_async_copy(ones, acc.at[eb.at[4, 1]], sems[1]).wait()
    plsc.subcore_barrier()
    pltpu.sync_copy(acc.at[pl.ds(s * ROWS_PT, ROWS_PT)],
                    out_hbm.at[c, pl.ds(s * ROWS_PT, ROWS_PT)])


@functools.partial(
    pl.kernel,
    out_type=jax.ShapeDtypeStruct((NC, N_PAD, F), jnp.float32),
    mesh=_mesh(),
    scratch_types=[
        pltpu.VMEM_SHARED((N_PAD, F), jnp.float32),
        pltpu.VMEM((_NEB, 2, CHUNK), jnp.int32),
        pltpu.VMEM((CHUNK, F), jnp.float32),
        pltpu.VMEM((CHUNK, F), jnp.float32),
        pltpu.VMEM((CHUNK, F), jnp.float32),
    ] + [pltpu.SemaphoreType.DMA] * (_NEB + 2 * _NRW),
)
def _agg_kernel(y_hbm, e_hbm, z128_hbm, out_hbm, acc, eb,
                rows0, rows1, rows2,
                si0, si1, si2, si3, si4, si5,
                sg0, sg1, sg2, ss0, ss1, ss2):
    c = lax.axis_index("c")
    s = lax.axis_index("s")
    wid = s * NC + c
    rows = (rows0, rows1, rows2)
    semi = (si0, si1, si2, si3, si4, si5)
    semg = (sg0, sg1, sg2)
    sems = (ss0, ss1, ss2)

    for j in range(4):
        pltpu.async_copy(e_hbm.at[wid, j], eb.at[j], semi[j])
    pltpu.sync_copy(z128_hbm, acc.at[pl.ds(s * ROWS_PT, ROWS_PT)])
    pltpu.make_async_copy(e_hbm.at[wid, 0], eb.at[0], semi[0]).wait()
    pltpu.async_copy(y_hbm.at[eb.at[0, 0]], rows0, semg[0])
    plsc.subcore_barrier()

    def step(j, m, wait_sc, do_pref, do_g):
        e, e1, e4 = m % _NEB, (m + 1) % _NEB, (m + 4) % _NEB
        r, r1 = m % _NRW, (m + 1) % _NRW
        if wait_sc:
            pltpu.make_async_copy(rows[r1], acc.at[eb.at[e4, 1]],
                                  sems[r1]).wait()
        if do_pref:
            pltpu.async_copy(e_hbm.at[wid, j + 4], eb.at[e4], semi[e4])
        if do_g:
            pltpu.make_async_copy(e_hbm.at[wid, j + 1], eb.at[e1],
                                  semi[e1]).wait()
            pltpu.async_copy(y_hbm.at[eb.at[e1, 0]], rows[r1], semg[r1])
        pltpu.make_async_copy(y_hbm.at[eb.at[e, 0]], rows[r], semg[r]).wait()
        pltpu.make_async_copy(rows[r], acc.at[eb.at[e, 1]],
                              sems[r]).start(add=True)

    step(0, 0, False, True, True)
    step(1, 1, False, True, True)
    for m in range(2, 6):
        step(m, m, True, True, True)

    def body(i, carry):
        base = 6 * i
        for m in range(6):
            step(base + m, m, True, True, True)
        return carry

    lax.fori_loop(1, (NCHUNK - 5) // 6, body, 0)
    step(NCHUNK - 5, 0, True, True, True)
    step(NCHUNK - 4, 1, True, False, True)
    step(NCHUNK - 3, 2, True, False, True)
    step(NCHUNK - 2, 3, True, False, True)
    step(NCHUNK - 1, 4, True, False, False)
    pltpu.make_async_copy(rows[0], acc.at[eb.at[3, 1]], sems[0]).wait()
    pltpu.make_async_copy(rows[1], acc.at[eb.at[4, 1]], sems[1]).wait()
    plsc.subcore_barrier()
    pltpu.sync_copy(acc.at[pl.ds(s * ROWS_PT, ROWS_PT)],
                    out_hbm.at[c, pl.ds(s * ROWS_PT, ROWS_PT)])


_BM = 1000


def _dis_from(deg_ref):
    deg = deg_ref[0, :, 0:1] + deg_ref[1, :, 0:1] + 1.0
    return lax.rsqrt(deg)


def _xw_body(x_ref, w1_ref, xw_ref):
    xw_ref[...] = jnp.dot(x_ref[...], w1_ref[...],
                          preferred_element_type=jnp.float32)


def _scale_body(deg_ref, x_ref, y_ref):
    dis = _dis_from(deg_ref)
    y_ref[...] = x_ref[...] * dis


def _layer1_body(deg_ref, s1_ref, xw_ref, b1_ref, w2_ref, h_ref, y2_ref):
    dis = _dis_from(deg_ref)
    p = dis * (s1_ref[0] + s1_ref[1]) + (dis * dis) * xw_ref[...]
    h = jnp.maximum(p + b1_ref[...], 0.0)
    hw = jnp.dot(h, w2_ref[...], preferred_element_type=jnp.float32)
    h_ref[...] = hw
    y2_ref[...] = hw * dis


def _layer2_body(deg_ref, s2_ref, hw_ref, b2_ref, mu_ref, sig_ref):
    dis = _dis_from(deg_ref)
    out = dis * (s2_ref[0] + s2_ref[1]) + (dis * dis) * hw_ref[...]
    out = out + b2_ref[...]
    mu_ref[...] = out[:, :F_OUT]
    sig_ref[...] = out[:, F_OUT:]


def _deg_spec():
    return pl.BlockSpec((2, _BM, F), lambda i: (0, i, 0))


def _row_spec(w=F):
    return pl.BlockSpec((_BM, w), lambda i: (i, 0))


def _part_spec(w=F):
    return pl.BlockSpec((2, _BM, w), lambda i: (0, i, 0))


def _full_spec(r, c):
    return pl.BlockSpec((r, c), lambda i: (0, 0))


_xw = pl.pallas_call(
    _xw_body,
    grid=(N // _BM,),
    in_specs=[_row_spec(), _full_spec(F, F)],
    out_specs=_row_spec(),
    out_shape=jax.ShapeDtypeStruct((N, F), jnp.float32),
)

_scale = pl.pallas_call(
    _scale_body,
    grid=(N // _BM,),
    in_specs=[_deg_spec(), _row_spec()],
    out_specs=_row_spec(),
    out_shape=jax.ShapeDtypeStruct((N, F), jnp.float32),
)

_layer1 = pl.pallas_call(
    _layer1_body,
    grid=(N // _BM,),
    in_specs=[_deg_spec(), _part_spec(), _row_spec(),
              _full_spec(1, F), _full_spec(F, 2 * F_OUT)],
    out_specs=[_row_spec(), _row_spec()],
    out_shape=[jax.ShapeDtypeStruct((N, F), jnp.float32),
               jax.ShapeDtypeStruct((N, F), jnp.float32)],
)

_layer2 = pl.pallas_call(
    _layer2_body,
    grid=(N // _BM,),
    in_specs=[_deg_spec(), _part_spec(), _row_spec(),
              _full_spec(1, 2 * F_OUT)],
    out_specs=[_row_spec(F_OUT), _row_spec(F_OUT)],
    out_shape=[jax.ShapeDtypeStruct((N, F_OUT), jnp.float32),
               jax.ShapeDtypeStruct((N, F_OUT), jnp.float32)],
)


def kernel(x, edge_index, W1, b1, Wmu, bmu, Wsig, bsig):
    z128 = jnp.zeros((ROWS_PT, F), jnp.float32)
    ar = jnp.arange(PAD_E, dtype=jnp.int32)
    pad_src = jnp.broadcast_to((ar * 37) % N, (NW, PAD_E))
    pad_dst = jnp.broadcast_to(N + ar % (N_PAD - N), (NW, PAD_E))
    pads = jnp.stack([pad_src, pad_dst])
    ei = jnp.concatenate([edge_index.reshape(2, NW, EPW), pads], axis=2)
    e3 = ei.reshape(2, NW, NCHUNK, CHUNK).transpose(1, 2, 0, 3)

    ones128 = jnp.ones((CHUNK, F), jnp.float32)
    w2 = jnp.concatenate([Wmu, Wsig], axis=1)
    b2 = jnp.concatenate([bmu, bsig]).reshape(1, 2 * F_OUT)
    xw = _xw(x, W1)
    degp = _deg_kernel(e3, z128, ones128)
    y1 = _scale(degp, xw)
    s1 = _agg_kernel(y1, e3, z128)
    hw, y2 = _layer1(degp, s1, xw, b1.reshape(1, F), w2)
    s2 = _agg_kernel(y2, e3, z128)
    mu, sig = _layer2(degp, s2, hw, b2)
    return mu, sig
